# Initial kernel scaffold; baseline (speedup 1.0000x reference)
import jax
import jax.numpy as jnp
from jax.experimental import pallas as pl


def kernel(x, edge_index, Wk, bk, Wq, bq, Wv, bv, Wa, ba, rel_att, rel_msg, rel_pri, skip):
    raise NotImplementedError("write your pallas kernel here")


# speedup vs baseline: 17.7381x; 17.7381x over previous
import functools

import jax
import jax.numpy as jnp
import numpy as np
from jax import lax
from jax.experimental import pallas as pl
from jax.experimental.pallas import tpu as pltpu
from jax.experimental.pallas import tpu_sc as plsc

N = 10000
E = 320000
D = 128
H = 8
DK = D // H

NC = 2
NS = 16
NW = NC * NS
EPT = E // NW
CHUNK = 80
NCHUNK = EPT // CHUNK
SUB = CHUNK // 16
ROWS_PT = 624
TAIL = N - NS * ROWS_PT

NPN = 640
DROW = NPN // 16
DSZ = NPN * H

_mesh = plsc.VectorSubcoreMesh(
    core_axis_name="c", subcore_axis_name="s", num_cores=NC, num_subcores=NS
)
_sc_params = pltpu.CompilerParams(needs_layout_passes=False)


def _copy_rows(src, dst, sid):
    pltpu.sync_copy(src.at[pl.ds(sid * ROWS_PT, ROWS_PT)],
                    dst.at[pl.ds(sid * ROWS_PT, ROWS_PT)])

    @pl.when(sid == NS - 1)
    def _():
        pltpu.sync_copy(src.at[pl.ds(NS * ROWS_PT, TAIL)],
                        dst.at[pl.ds(NS * ROWS_PT, TAIL)])


def _iota16():
    return lax.iota(jnp.int32, 16)


def _full16(v):
    return jnp.full((16,), v, jnp.int32)


@functools.partial(
    pl.kernel,
    mesh=_mesh,
    out_type=[
        jax.ShapeDtypeStruct((E // CHUNK, H * CHUNK), jnp.float32),
        jax.ShapeDtypeStruct((NW * NPN, D), jnp.float32),
    ],
    scratch_types=[
        pltpu.VMEM((CHUNK,), jnp.int32),
        pltpu.VMEM((CHUNK,), jnp.int32),
        pltpu.VMEM((H * CHUNK + 16,), jnp.float32),
        pltpu.VMEM((NPN * 16 * H,), jnp.float32),
        pltpu.VMEM((DROW, D), jnp.float32),
        pltpu.SemaphoreType.DMA,
        pltpu.SemaphoreType.DMA,
    ],
    compiler_params=_sc_params,
)
def _sc_scores(q_hbm, k_hbm, src_hbm, dst_hbm,
               p_hbm, den_hbm,
               sidx, didx, pblkT, dacc, dstage, sem1, sem2):
    cid = lax.axis_index("c")
    sid = lax.axis_index("s")
    wid = sid * NC + cid

    io = _iota16()
    zv = jnp.zeros((16,), jnp.float32)

    def zero_body(i, _):
        dacc[pl.ds(i * 16, 16)] = zv
        return 0

    lax.fori_loop(0, NPN * 16 * H // 16, zero_body, 0)
    pblkT[pl.ds(H * CHUNK, 16)] = zv

    def scoped(qblk, kblk):
        def chunk_body(c, _):
            base = wid * EPT + c * CHUNK
            pltpu.sync_copy(src_hbm.at[pl.ds(base, CHUNK)], sidx)
            pltpu.sync_copy(dst_hbm.at[pl.ds(base, CHUNK)], didx)
            cp_q = pltpu.async_copy(q_hbm.at[didx], qblk, sem1)
            cp_k = pltpu.async_copy(k_hbm.at[sidx], kblk, sem2)
            cp_q.wait()
            cp_k.wait()
            for b in range(SUB):
                rows = _full16(b * 16) + io
                for h in range(H):
                    s = jnp.zeros((16,), jnp.float32)
                    for dk in range(DK):
                        col = _full16(h * DK + dk)
                        qc = plsc.load_gather(qblk, [rows, col])
                        kc = plsc.load_gather(kblk, [rows, col])
                        s = s + qc * kc
                    pblkT[pl.ds(h * CHUNK + b * 16, 16)] = jnp.exp(s)
            for b in range(SUB):
                dv = didx[pl.ds(b * 16, 16)] * H
                for l in range(16):
                    e = b * 16 + l
                    prow = plsc.load_gather(
                        pblkT, [jnp.where(io < H, io * CHUNK + e, H * CHUNK)])
                    d8 = dv[l]
                    dacc[pl.ds(d8, 16)] = dacc[pl.ds(d8, 16)] + prow
            pltpu.sync_copy(pblkT.at[pl.ds(0, H * CHUNK)],
                            p_hbm.at[wid * NCHUNK + c])
            return 0

        lax.fori_loop(0, NCHUNK, chunk_body, 0)

    pl.run_scoped(scoped,
                  pltpu.VMEM((CHUNK, D), jnp.float32),
                  pltpu.VMEM((CHUNK, D), jnp.float32))

    def dump_piece(pc, _):
        def row_body(r, _):
            for j in range(8):
                dstage[r, pl.ds(j * 16, 16)] = (
                    dacc[pl.ds((pc * DROW + r) * D + j * 16, 16)])
            return 0

        lax.fori_loop(0, DROW, row_body, 0)
        pltpu.sync_copy(dstage,
                        den_hbm.at[pl.ds(wid * NPN + pc * DROW, DROW)])
        return 0

    lax.fori_loop(0, 16, dump_piece, 0)


@functools.partial(
    pl.kernel,
    mesh=_mesh,
    out_type=[
        jax.ShapeDtypeStruct((N, D), jnp.float32),
        jax.ShapeDtypeStruct((N, D), jnp.float32),
    ],
    scratch_types=[
        pltpu.VMEM((CHUNK,), jnp.int32),
        pltpu.VMEM((CHUNK,), jnp.int32),
        pltpu.VMEM((CHUNK,), jnp.int32),
        pltpu.VMEM((CHUNK, D), jnp.float32),
        pltpu.VMEM((H * CHUNK,), jnp.float32),
        pltpu.VMEM((H * CHUNK,), jnp.float32),
        pltpu.VMEM((CHUNK, D), jnp.float32),
        pltpu.VMEM((DROW, D), jnp.float32),
        pltpu.VMEM((DROW, D), jnp.float32),
        pltpu.VMEM_SHARED((NS * DROW, D), jnp.float32),
        pltpu.VMEM_SHARED((N, D), jnp.float32),
        pltpu.SemaphoreType.DMA,
        pltpu.SemaphoreType.DMA,
    ],
    compiler_params=_sc_params,
)
def _sc_aggregate(v_hbm, src_hbm, dst_hbm, p_hbm, den_hbm, znd_hbm,
                  agg0_hbm, agg1_hbm,
                  sidx, didx, didx2, vblk, pblkT, attnT, msgblk, acc, stg,
                  rden_sh, agg_sh, sem1, sem2):
    cid = lax.axis_index("c")
    sid = lax.axis_index("s")
    wid = sid * NC + cid

    _copy_rows(znd_hbm, agg_sh, sid)

    io = _iota16()

    rowbase = sid * DROW

    def zero_body(r, _):
        for j in range(8):
            acc[r, pl.ds(j * 16, 16)] = jnp.zeros((16,), jnp.float32)
        return 0

    lax.fori_loop(0, DROW, zero_body, 0)

    def add_tile(t, _):
        pltpu.sync_copy(den_hbm.at[pl.ds(t * NPN + rowbase, DROW)], stg)

        def row_body(r, _):
            for j in range(8):
                sl = pl.ds(j * 16, 16)
                acc[r, sl] = acc[r, sl] + stg[r, sl]
            return 0

        lax.fori_loop(0, DROW, row_body, 0)
        return 0

    lax.fori_loop(0, NW, add_tile, 0)

    def rcp_body(r, _):
        for j in range(8):
            sl = pl.ds(j * 16, 16)
            acc[r, sl] = 1.0 / acc[r, sl]
        return 0

    lax.fori_loop(0, DROW, rcp_body, 0)
    pltpu.sync_copy(acc, rden_sh.at[pl.ds(rowbase, DROW)])
    plsc.subcore_barrier()

    def scoped(rdnblk):
        def chunk_body(c, _):
            base = wid * EPT + c * CHUNK
            pltpu.sync_copy(src_hbm.at[pl.ds(base, CHUNK)], sidx)
            pltpu.sync_copy(dst_hbm.at[pl.ds(base, CHUNK)], didx)
            pltpu.sync_copy(p_hbm.at[wid * NCHUNK + c], pblkT)
            cp_v = pltpu.async_copy(v_hbm.at[sidx], vblk, sem1)
            for b in range(SUB):
                didx2[pl.ds(b * 16, 16)] = (
                    lax.shift_right_logical(didx[pl.ds(b * 16, 16)], 4))
            cp_r = pltpu.async_copy(rden_sh.at[didx2], rdnblk, sem2)
            cp_v.wait()
            cp_r.wait()
            for b in range(SUB):
                rows = _full16(b * 16) + io
                coff = (didx[pl.ds(b * 16, 16)] & 15) * H
                for h in range(H):
                    pcol = pblkT[pl.ds(h * CHUNK + b * 16, 16)]
                    rd = plsc.load_gather(rdnblk, [rows, coff + h])
                    attnT[pl.ds(h * CHUNK + b * 16, 16)] = pcol * rd
            hidx = (io & 7) * CHUNK
            for e in range(CHUNK):
                arep = plsc.load_gather(attnT, [hidx + e])
                for j in range(H):
                    msgblk[e, pl.ds(j * 16, 16)] = (
                        vblk[e, pl.ds(j * 16, 16)] * arep)
            pltpu.sync_copy(msgblk, agg_sh.at[didx], add=True)
            return 0

        lax.fori_loop(0, NCHUNK, chunk_body, 0)

    pl.run_scoped(scoped, pltpu.VMEM((CHUNK, D), jnp.float32))
    plsc.subcore_barrier()

    @pl.when(cid == 0)
    def _():
        _copy_rows(agg_sh, agg0_hbm, sid)

    @pl.when(cid == 1)
    def _():
        _copy_rows(agg_sh, agg1_hbm, sid)


_RB = 1000


def _qkv_body(x_ref, w_ref, b_ref, q_ref, k_ref, v_ref):
    y = jnp.dot(x_ref[...], w_ref[...], preferred_element_type=jnp.float32)
    y = y + b_ref[...]
    q_ref[...] = y[:, :D]
    k_ref[...] = y[:, D:2 * D]
    v_ref[...] = y[:, 2 * D:]


_qkv_call = pl.pallas_call(
    _qkv_body,
    grid=(N // _RB,),
    in_specs=[
        pl.BlockSpec((_RB, D), lambda i: (i, 0)),
        pl.BlockSpec((D, 3 * D), lambda i: (0, 0)),
        pl.BlockSpec((1, 3 * D), lambda i: (0, 0)),
    ],
    out_specs=[
        pl.BlockSpec((_RB, D), lambda i: (i, 0)),
        pl.BlockSpec((_RB, D), lambda i: (i, 0)),
        pl.BlockSpec((_RB, D), lambda i: (i, 0)),
    ],
    out_shape=[
        jax.ShapeDtypeStruct((N, D), jnp.float32),
        jax.ShapeDtypeStruct((N, D), jnp.float32),
        jax.ShapeDtypeStruct((N, D), jnp.float32),
    ],
)


def _final_body(a0_ref, a1_ref, x_ref, w_ref, b_ref, al_ref, o_ref):
    agg = a0_ref[...] + a1_ref[...]
    trans = jnp.dot(agg, w_ref[...], preferred_element_type=jnp.float32)
    trans = trans + b_ref[...]
    alpha = al_ref[0, 0]
    o_ref[...] = trans * alpha + x_ref[...] * (1.0 - alpha)


_final_call = pl.pallas_call(
    _final_body,
    grid=(N // _RB,),
    in_specs=[
        pl.BlockSpec((_RB, D), lambda i: (i, 0)),
        pl.BlockSpec((_RB, D), lambda i: (i, 0)),
        pl.BlockSpec((_RB, D), lambda i: (i, 0)),
        pl.BlockSpec((D, D), lambda i: (0, 0)),
        pl.BlockSpec((1, D), lambda i: (0, 0)),
        pl.BlockSpec((1, 1), lambda i: (0, 0)),
    ],
    out_specs=pl.BlockSpec((_RB, D), lambda i: (i, 0)),
    out_shape=jax.ShapeDtypeStruct((N, D), jnp.float32),
)


def kernel(x, edge_index, Wk, bk, Wq, bq, Wv, bv, Wa, ba, rel_att, rel_msg,
           rel_pri, skip):
    src = edge_index[0]
    dst = edge_index[1]

    scale = rel_pri[0] / np.sqrt(DK)
    Wq_f = (Wq.reshape(D, H, DK) * scale[None, :, None]).reshape(D, D)
    bq_f = (bq.reshape(H, DK) * scale[:, None]).reshape(D)
    Wk_f = jnp.einsum('dhj,hjk->dhk', Wk.reshape(D, H, DK), rel_att[0]).reshape(D, D)
    bk_f = jnp.einsum('hj,hjk->hk', bk.reshape(H, DK), rel_att[0]).reshape(D)
    Wv_f = jnp.einsum('dhj,hjk->dhk', Wv.reshape(D, H, DK), rel_msg[0]).reshape(D, D)
    bv_f = jnp.einsum('hj,hjk->hk', bv.reshape(H, DK), rel_msg[0]).reshape(D)
    perm = np.array([(j % H) * DK + j // H for j in range(D)])
    Wcat = jnp.concatenate([Wq_f, Wk_f, Wv_f[:, perm]], axis=1)
    bcat = jnp.concatenate([bq_f, bk_f, bv_f[perm]]).reshape(1, 3 * D)

    q, k, v = _qkv_call(x, Wcat, bcat)

    znd = jnp.zeros((N, D), jnp.float32)

    p, den = _sc_scores(q, k, src, dst)
    agg0, agg1 = _sc_aggregate(v, src, dst, p, den, znd)

    alpha = jax.nn.sigmoid(skip[0]).reshape(1, 1)
    return _final_call(agg0, agg1, x, Wa[perm, :], ba.reshape(1, D), alpha)


# speedup vs baseline: 18.7443x; 1.0567x over previous
import functools

import jax
import jax.numpy as jnp
import numpy as np
from jax import lax
from jax.experimental import pallas as pl
from jax.experimental.pallas import tpu as pltpu
from jax.experimental.pallas import tpu_sc as plsc

N = 10000
E = 320000
D = 128
H = 8
DK = D // H

NC = 2
NS = 16
NW = NC * NS
CHUNK = 128
ECHUNKS = E // CHUNK
NCH_MAIN = ECHUNKS // NW
TAILC = ECHUNKS - NW * NCH_MAIN
SUB = CHUNK // 16
ROWS_PT = 624
TAIL = N - NS * ROWS_PT

NPN = 640
DROW = NPN // 16
DSZ = NPN * H

_mesh = plsc.VectorSubcoreMesh(
    core_axis_name="c", subcore_axis_name="s", num_cores=NC, num_subcores=NS
)
_sc_params = pltpu.CompilerParams(needs_layout_passes=False)


def _copy_rows(src, dst, sid):
    pltpu.sync_copy(src.at[pl.ds(sid * ROWS_PT, ROWS_PT)],
                    dst.at[pl.ds(sid * ROWS_PT, ROWS_PT)])

    @pl.when(sid == NS - 1)
    def _():
        pltpu.sync_copy(src.at[pl.ds(NS * ROWS_PT, TAIL)],
                        dst.at[pl.ds(NS * ROWS_PT, TAIL)])


def _iota16():
    return lax.iota(jnp.int32, 16)


def _full16(v):
    return jnp.full((16,), v, jnp.int32)


@functools.partial(
    pl.kernel,
    mesh=_mesh,
    out_type=[
        jax.ShapeDtypeStruct((ECHUNKS, H * CHUNK), jnp.float32),
        jax.ShapeDtypeStruct((NW * NPN, D), jnp.float32),
    ],
    scratch_types=[
        pltpu.VMEM((CHUNK,), jnp.int32),
        pltpu.VMEM((CHUNK,), jnp.int32),
        pltpu.VMEM((H * CHUNK + 16,), jnp.float32),
        pltpu.VMEM((NPN * 16 * H,), jnp.float32),
        pltpu.VMEM((DROW, D), jnp.float32),
        pltpu.SemaphoreType.DMA,
        pltpu.SemaphoreType.DMA,
    ],
    compiler_params=_sc_params,
)
def _sc_scores(q_hbm, k_hbm, src_hbm, dst_hbm,
               p_hbm, den_hbm,
               sidx, didx, pblkT, dacc, dstage, sem1, sem2):
    cid = lax.axis_index("c")
    sid = lax.axis_index("s")
    wid = sid * NC + cid

    io = _iota16()
    zv = jnp.zeros((16,), jnp.float32)

    def zero_body(i, _):
        dacc[pl.ds(i * 16, 16)] = zv
        return 0

    lax.fori_loop(0, NPN * 16 * H // 16, zero_body, 0)
    pblkT[pl.ds(H * CHUNK, 16)] = zv

    def scoped(qblk, kblk):
        def chunk_work(g):
            base = g * CHUNK
            pltpu.sync_copy(src_hbm.at[pl.ds(base, CHUNK)], sidx)
            pltpu.sync_copy(dst_hbm.at[pl.ds(base, CHUNK)], didx)
            cp_q = pltpu.async_copy(q_hbm.at[didx], qblk, sem1)
            cp_k = pltpu.async_copy(k_hbm.at[sidx], kblk, sem2)
            cp_q.wait()
            cp_k.wait()

            def sub_body(b, _):
                rows = _full16(0) + io + b * 16
                for h in range(H):
                    s = jnp.zeros((16,), jnp.float32)
                    for dk in range(DK):
                        col = _full16(h * DK + dk)
                        qc = plsc.load_gather(qblk, [rows, col])
                        kc = plsc.load_gather(kblk, [rows, col])
                        s = s + qc * kc
                    pblkT[pl.ds(h * CHUNK + b * 16, 16)] = jnp.exp(s)
                dv = didx[pl.ds(b * 16, 16)] * H
                for l in range(16):
                    prow = plsc.load_gather(
                        pblkT,
                        [jnp.where(io < H, io * CHUNK + b * 16 + l, H * CHUNK)])
                    d8 = dv[l]
                    dacc[pl.ds(d8, 16)] = dacc[pl.ds(d8, 16)] + prow
                return 0

            lax.fori_loop(0, SUB, sub_body, 0)
            pltpu.sync_copy(pblkT.at[pl.ds(0, H * CHUNK)], p_hbm.at[g])

        def chunk_body(c, _):
            chunk_work(c * NW + wid)
            return 0

        lax.fori_loop(0, NCH_MAIN, chunk_body, 0)

        @pl.when(wid < TAILC)
        def _():
            chunk_work(NW * NCH_MAIN + wid)

    pl.run_scoped(scoped,
                  pltpu.VMEM((CHUNK, D), jnp.float32),
                  pltpu.VMEM((CHUNK, D), jnp.float32))

    def dump_piece(pc, _):
        def row_body(r, _):
            for j in range(8):
                dstage[r, pl.ds(j * 16, 16)] = (
                    dacc[pl.ds((pc * DROW + r) * D + j * 16, 16)])
            return 0

        lax.fori_loop(0, DROW, row_body, 0)
        pltpu.sync_copy(dstage,
                        den_hbm.at[pl.ds(wid * NPN + pc * DROW, DROW)])
        return 0

    lax.fori_loop(0, 16, dump_piece, 0)


@functools.partial(
    pl.kernel,
    mesh=_mesh,
    out_type=[
        jax.ShapeDtypeStruct((N, D), jnp.float32),
        jax.ShapeDtypeStruct((N, D), jnp.float32),
    ],
    scratch_types=[
        pltpu.VMEM((CHUNK,), jnp.int32),
        pltpu.VMEM((CHUNK,), jnp.int32),
        pltpu.VMEM((CHUNK,), jnp.int32),
        pltpu.VMEM((H * CHUNK,), jnp.float32),
        pltpu.VMEM((H * CHUNK,), jnp.float32),
        pltpu.VMEM((DROW, D), jnp.float32),
        pltpu.VMEM((DROW, D), jnp.float32),
        pltpu.VMEM_SHARED((NS * DROW, D), jnp.float32),
        pltpu.VMEM_SHARED((N, D), jnp.float32),
        pltpu.SemaphoreType.DMA,
        pltpu.SemaphoreType.DMA,
    ],
    compiler_params=_sc_params,
)
def _sc_aggregate(v_hbm, src_hbm, dst_hbm, p_hbm, den_hbm, znd_hbm,
                  agg0_hbm, agg1_hbm,
                  sidx, didx, didx2, pblkT, attnT, acc, stg,
                  rden_sh, agg_sh, sem1, sem2):
    cid = lax.axis_index("c")
    sid = lax.axis_index("s")
    wid = sid * NC + cid

    _copy_rows(znd_hbm, agg_sh, sid)

    io = _iota16()

    rowbase = sid * DROW

    def zero_body(r, _):
        for j in range(8):
            acc[r, pl.ds(j * 16, 16)] = jnp.zeros((16,), jnp.float32)
        return 0

    lax.fori_loop(0, DROW, zero_body, 0)

    def add_tile(t, _):
        pltpu.sync_copy(den_hbm.at[pl.ds(t * NPN + rowbase, DROW)], stg)

        def row_body(r, _):
            for j in range(8):
                sl = pl.ds(j * 16, 16)
                acc[r, sl] = acc[r, sl] + stg[r, sl]
            return 0

        lax.fori_loop(0, DROW, row_body, 0)
        return 0

    lax.fori_loop(0, NW, add_tile, 0)

    def rcp_body(r, _):
        for j in range(8):
            sl = pl.ds(j * 16, 16)
            acc[r, sl] = 1.0 / acc[r, sl]
        return 0

    lax.fori_loop(0, DROW, rcp_body, 0)
    pltpu.sync_copy(acc, rden_sh.at[pl.ds(rowbase, DROW)])
    plsc.subcore_barrier()

    def scoped(vblk, rdnblk):
        def chunk_work(g):
            base = g * CHUNK
            pltpu.sync_copy(src_hbm.at[pl.ds(base, CHUNK)], sidx)
            pltpu.sync_copy(dst_hbm.at[pl.ds(base, CHUNK)], didx)
            pltpu.sync_copy(p_hbm.at[g], pblkT)
            cp_v = pltpu.async_copy(v_hbm.at[sidx], vblk, sem1)

            def didx2_body(b, _):
                didx2[pl.ds(b * 16, 16)] = (
                    lax.shift_right_logical(didx[pl.ds(b * 16, 16)], 4))
                return 0

            lax.fori_loop(0, SUB, didx2_body, 0)
            cp_r = pltpu.async_copy(rden_sh.at[didx2], rdnblk, sem2)
            cp_v.wait()
            cp_r.wait()

            def sub_body(b, _):
                rows = _full16(0) + io + b * 16
                coff = (didx[pl.ds(b * 16, 16)] & 15) * H
                for h in range(H):
                    pcol = pblkT[pl.ds(h * CHUNK + b * 16, 16)]
                    rd = plsc.load_gather(rdnblk, [rows, coff + h])
                    attnT[pl.ds(h * CHUNK + b * 16, 16)] = pcol * rd
                hidx = (io & 7) * CHUNK + b * 16
                for l in range(16):
                    e = b * 16 + l
                    arep = plsc.load_gather(attnT, [hidx + l])
                    for j in range(H):
                        vblk[e, pl.ds(j * 16, 16)] = (
                            vblk[e, pl.ds(j * 16, 16)] * arep)
                return 0

            lax.fori_loop(0, SUB, sub_body, 0)
            pltpu.sync_copy(vblk, agg_sh.at[didx], add=True)

        def chunk_body(c, _):
            chunk_work(c * NW + wid)
            return 0

        lax.fori_loop(0, NCH_MAIN, chunk_body, 0)

        @pl.when(wid < TAILC)
        def _():
            chunk_work(NW * NCH_MAIN + wid)

    pl.run_scoped(scoped,
                  pltpu.VMEM((CHUNK, D), jnp.float32),
                  pltpu.VMEM((CHUNK, D), jnp.float32))
    plsc.subcore_barrier()

    @pl.when(cid == 0)
    def _():
        _copy_rows(agg_sh, agg0_hbm, sid)

    @pl.when(cid == 1)
    def _():
        _copy_rows(agg_sh, agg1_hbm, sid)


_RB = 1000


def _qkv_body(x_ref, w_ref, b_ref, q_ref, k_ref, v_ref):
    y = jnp.dot(x_ref[...], w_ref[...], preferred_element_type=jnp.float32)
    y = y + b_ref[...]
    q_ref[...] = y[:, :D]
    k_ref[...] = y[:, D:2 * D]
    v_ref[...] = y[:, 2 * D:]


_qkv_call = pl.pallas_call(
    _qkv_body,
    grid=(N // _RB,),
    in_specs=[
        pl.BlockSpec((_RB, D), lambda i: (i, 0)),
        pl.BlockSpec((D, 3 * D), lambda i: (0, 0)),
        pl.BlockSpec((1, 3 * D), lambda i: (0, 0)),
    ],
    out_specs=[
        pl.BlockSpec((_RB, D), lambda i: (i, 0)),
        pl.BlockSpec((_RB, D), lambda i: (i, 0)),
        pl.BlockSpec((_RB, D), lambda i: (i, 0)),
    ],
    out_shape=[
        jax.ShapeDtypeStruct((N, D), jnp.float32),
        jax.ShapeDtypeStruct((N, D), jnp.float32),
        jax.ShapeDtypeStruct((N, D), jnp.float32),
    ],
)


def _final_body(a0_ref, a1_ref, x_ref, w_ref, b_ref, al_ref, o_ref):
    agg = a0_ref[...] + a1_ref[...]
    trans = jnp.dot(agg, w_ref[...], preferred_element_type=jnp.float32)
    trans = trans + b_ref[...]
    alpha = al_ref[0, 0]
    o_ref[...] = trans * alpha + x_ref[...] * (1.0 - alpha)


_final_call = pl.pallas_call(
    _final_body,
    grid=(N // _RB,),
    in_specs=[
        pl.BlockSpec((_RB, D), lambda i: (i, 0)),
        pl.BlockSpec((_RB, D), lambda i: (i, 0)),
        pl.BlockSpec((_RB, D), lambda i: (i, 0)),
        pl.BlockSpec((D, D), lambda i: (0, 0)),
        pl.BlockSpec((1, D), lambda i: (0, 0)),
        pl.BlockSpec((1, 1), lambda i: (0, 0)),
    ],
    out_specs=pl.BlockSpec((_RB, D), lambda i: (i, 0)),
    out_shape=jax.ShapeDtypeStruct((N, D), jnp.float32),
)


def kernel(x, edge_index, Wk, bk, Wq, bq, Wv, bv, Wa, ba, rel_att, rel_msg,
           rel_pri, skip):
    src = edge_index[0]
    dst = edge_index[1]

    scale = rel_pri[0] / np.sqrt(DK)
    Wq_f = (Wq.reshape(D, H, DK) * scale[None, :, None]).reshape(D, D)
    bq_f = (bq.reshape(H, DK) * scale[:, None]).reshape(D)
    Wk_f = jnp.einsum('dhj,hjk->dhk', Wk.reshape(D, H, DK), rel_att[0]).reshape(D, D)
    bk_f = jnp.einsum('hj,hjk->hk', bk.reshape(H, DK), rel_att[0]).reshape(D)
    Wv_f = jnp.einsum('dhj,hjk->dhk', Wv.reshape(D, H, DK), rel_msg[0]).reshape(D, D)
    bv_f = jnp.einsum('hj,hjk->hk', bv.reshape(H, DK), rel_msg[0]).reshape(D)
    perm = np.array([(j % H) * DK + j // H for j in range(D)])
    Wcat = jnp.concatenate([Wq_f, Wk_f, Wv_f[:, perm]], axis=1)
    bcat = jnp.concatenate([bq_f, bk_f, bv_f[perm]]).reshape(1, 3 * D)

    q, k, v = _qkv_call(x, Wcat, bcat)

    znd = jnp.zeros((N, D), jnp.float32)

    p, den = _sc_scores(q, k, src, dst)
    agg0, agg1 = _sc_aggregate(v, src, dst, p, den, znd)

    alpha = jax.nn.sigmoid(skip[0]).reshape(1, 1)
    return _final_call(agg0, agg1, x, Wa[perm, :], ba.reshape(1, D), alpha)


# speedup vs baseline: 19.4409x; 1.0372x over previous
import functools

import jax
import jax.numpy as jnp
import numpy as np
from jax import lax
from jax.experimental import pallas as pl
from jax.experimental.pallas import tpu as pltpu
from jax.experimental.pallas import tpu_sc as plsc

N = 10000
E = 320000
D = 128
H = 8
DK = D // H

NC = 2
NS = 16
NW = NC * NS
CHUNK = 128
ECHUNKS = E // CHUNK
NCH_MAIN = ECHUNKS // NW
TAILC = ECHUNKS - NW * NCH_MAIN
SUB = CHUNK // 16
ROWS_PT = 624
TAIL = N - NS * ROWS_PT

NPN = 640
DROW = NPN // 16
DSZ = NPN * H

_mesh = plsc.VectorSubcoreMesh(
    core_axis_name="c", subcore_axis_name="s", num_cores=NC, num_subcores=NS
)
_sc_params = pltpu.CompilerParams(needs_layout_passes=False)


def _copy_rows(src, dst, sid):
    pltpu.sync_copy(src.at[pl.ds(sid * ROWS_PT, ROWS_PT)],
                    dst.at[pl.ds(sid * ROWS_PT, ROWS_PT)])

    @pl.when(sid == NS - 1)
    def _():
        pltpu.sync_copy(src.at[pl.ds(NS * ROWS_PT, TAIL)],
                        dst.at[pl.ds(NS * ROWS_PT, TAIL)])


def _iota16():
    return lax.iota(jnp.int32, 16)


def _full16(v):
    return jnp.full((16,), v, jnp.int32)


@functools.partial(
    pl.kernel,
    mesh=_mesh,
    out_type=[
        jax.ShapeDtypeStruct((ECHUNKS, H * CHUNK), jnp.float32),
        jax.ShapeDtypeStruct((NW * NPN, D), jnp.float32),
    ],
    scratch_types=[
        pltpu.VMEM((CHUNK,), jnp.int32),
        pltpu.VMEM((CHUNK,), jnp.int32),
        pltpu.VMEM((H * CHUNK,), jnp.float32),
        pltpu.VMEM((NPN * 16 * H,), jnp.float32),
        pltpu.VMEM((DROW, D), jnp.float32),
        pltpu.SemaphoreType.DMA,
        pltpu.SemaphoreType.DMA,
    ],
    compiler_params=_sc_params,
)
def _sc_scores(q_hbm, k_hbm, src_hbm, dst_hbm,
               p_hbm, den_hbm,
               sidx, didx, pblkT, dacc, dstage, sem1, sem2):
    cid = lax.axis_index("c")
    sid = lax.axis_index("s")
    wid = sid * NC + cid

    io = _iota16()
    zv = jnp.zeros((16,), jnp.float32)

    def zero_body(i, _):
        dacc[pl.ds(i * 16, 16)] = zv
        return 0

    lax.fori_loop(0, NPN * 16 * H // 16, zero_body, 0)

    def scoped(qblk, kblk):
        def chunk_work(g):
            base = g * CHUNK
            pltpu.sync_copy(src_hbm.at[pl.ds(base, CHUNK)], sidx)
            pltpu.sync_copy(dst_hbm.at[pl.ds(base, CHUNK)], didx)
            cp_q = pltpu.async_copy(q_hbm.at[didx], qblk, sem1)
            cp_k = pltpu.async_copy(k_hbm.at[sidx], kblk, sem2)
            cp_q.wait()
            cp_k.wait()

            def sub_body(b, _):
                rows = _full16(0) + io + b * 16
                dv = didx[pl.ds(b * 16, 16)] * H
                for h in range(H):
                    s = jnp.zeros((16,), jnp.float32)
                    for dk in range(DK):
                        col = _full16(h * DK + dk)
                        qc = plsc.load_gather(qblk, [rows, col])
                        kc = plsc.load_gather(kblk, [rows, col])
                        s = s + qc * kc
                    pexp = jnp.exp(s)
                    pblkT[pl.ds(h * CHUNK + b * 16, 16)] = pexp
                    plsc.addupdate_scatter(dacc, [dv + h], pexp)
                return 0

            lax.fori_loop(0, SUB, sub_body, 0)
            pltpu.sync_copy(pblkT, p_hbm.at[g])

        def chunk_body(c, _):
            chunk_work(c * NW + wid)
            return 0

        lax.fori_loop(0, NCH_MAIN, chunk_body, 0)

        @pl.when(wid < TAILC)
        def _():
            chunk_work(NW * NCH_MAIN + wid)

    pl.run_scoped(scoped,
                  pltpu.VMEM((CHUNK, D), jnp.float32),
                  pltpu.VMEM((CHUNK, D), jnp.float32))

    def dump_piece(pc, _):
        def row_body(r, _):
            for j in range(8):
                dstage[r, pl.ds(j * 16, 16)] = (
                    dacc[pl.ds((pc * DROW + r) * D + j * 16, 16)])
            return 0

        lax.fori_loop(0, DROW, row_body, 0)
        pltpu.sync_copy(dstage,
                        den_hbm.at[pl.ds(wid * NPN + pc * DROW, DROW)])
        return 0

    lax.fori_loop(0, 16, dump_piece, 0)


@functools.partial(
    pl.kernel,
    mesh=_mesh,
    out_type=[
        jax.ShapeDtypeStruct((N, D), jnp.float32),
        jax.ShapeDtypeStruct((N, D), jnp.float32),
    ],
    scratch_types=[
        pltpu.VMEM((CHUNK,), jnp.int32),
        pltpu.VMEM((CHUNK,), jnp.int32),
        pltpu.VMEM((CHUNK,), jnp.int32),
        pltpu.VMEM((H * CHUNK,), jnp.float32),
        pltpu.VMEM((H * CHUNK,), jnp.float32),
        pltpu.VMEM((DROW, D), jnp.float32),
        pltpu.VMEM((DROW, D), jnp.float32),
        pltpu.VMEM_SHARED((NS * DROW, D), jnp.float32),
        pltpu.VMEM_SHARED((N, D), jnp.float32),
        pltpu.SemaphoreType.DMA,
        pltpu.SemaphoreType.DMA,
    ],
    compiler_params=_sc_params,
)
def _sc_aggregate(v_hbm, src_hbm, dst_hbm, p_hbm, den_hbm, znd_hbm,
                  agg0_hbm, agg1_hbm,
                  sidx, didx, didx2, pblkT, attnT, acc, stg,
                  rden_sh, agg_sh, sem1, sem2):
    cid = lax.axis_index("c")
    sid = lax.axis_index("s")
    wid = sid * NC + cid

    _copy_rows(znd_hbm, agg_sh, sid)

    io = _iota16()

    rowbase = sid * DROW

    def zero_body(r, _):
        for j in range(8):
            acc[r, pl.ds(j * 16, 16)] = jnp.zeros((16,), jnp.float32)
        return 0

    lax.fori_loop(0, DROW, zero_body, 0)

    def add_tile(t, _):
        pltpu.sync_copy(den_hbm.at[pl.ds(t * NPN + rowbase, DROW)], stg)

        def row_body(r, _):
            for j in range(8):
                sl = pl.ds(j * 16, 16)
                acc[r, sl] = acc[r, sl] + stg[r, sl]
            return 0

        lax.fori_loop(0, DROW, row_body, 0)
        return 0

    lax.fori_loop(0, NW, add_tile, 0)

    def rcp_body(r, _):
        for j in range(8):
            sl = pl.ds(j * 16, 16)
            acc[r, sl] = 1.0 / acc[r, sl]
        return 0

    lax.fori_loop(0, DROW, rcp_body, 0)
    pltpu.sync_copy(acc, rden_sh.at[pl.ds(rowbase, DROW)])
    plsc.subcore_barrier()

    def scoped(vblk, rdnblk):
        def chunk_work(g):
            base = g * CHUNK
            pltpu.sync_copy(src_hbm.at[pl.ds(base, CHUNK)], sidx)
            pltpu.sync_copy(dst_hbm.at[pl.ds(base, CHUNK)], didx)
            pltpu.sync_copy(p_hbm.at[g], pblkT)
            cp_v = pltpu.async_copy(v_hbm.at[sidx], vblk, sem1)

            def didx2_body(b, _):
                didx2[pl.ds(b * 16, 16)] = (
                    lax.shift_right_logical(didx[pl.ds(b * 16, 16)], 4))
                return 0

            lax.fori_loop(0, SUB, didx2_body, 0)
            cp_r = pltpu.async_copy(rden_sh.at[didx2], rdnblk, sem2)
            cp_v.wait()
            cp_r.wait()

            def sub_body(b, _):
                rows = _full16(0) + io + b * 16
                coff = (didx[pl.ds(b * 16, 16)] & 15) * H
                for h in range(H):
                    pcol = pblkT[pl.ds(h * CHUNK + b * 16, 16)]
                    rd = plsc.load_gather(rdnblk, [rows, coff + h])
                    attnT[pl.ds(h * CHUNK + b * 16, 16)] = pcol * rd
                hidx = (io & 7) * CHUNK + b * 16
                for l in range(16):
                    e = b * 16 + l
                    arep = plsc.load_gather(attnT, [hidx + l])
                    for j in range(H):
                        vblk[e, pl.ds(j * 16, 16)] = (
                            vblk[e, pl.ds(j * 16, 16)] * arep)
                return 0

            lax.fori_loop(0, SUB, sub_body, 0)
            pltpu.sync_copy(vblk, agg_sh.at[didx], add=True)

        def chunk_body(c, _):
            chunk_work(c * NW + wid)
            return 0

        lax.fori_loop(0, NCH_MAIN, chunk_body, 0)

        @pl.when(wid < TAILC)
        def _():
            chunk_work(NW * NCH_MAIN + wid)

    pl.run_scoped(scoped,
                  pltpu.VMEM((CHUNK, D), jnp.float32),
                  pltpu.VMEM((CHUNK, D), jnp.float32))
    plsc.subcore_barrier()

    @pl.when(cid == 0)
    def _():
        _copy_rows(agg_sh, agg0_hbm, sid)

    @pl.when(cid == 1)
    def _():
        _copy_rows(agg_sh, agg1_hbm, sid)


_RB = 1000


def _qkv_body(x_ref, w_ref, b_ref, q_ref, k_ref, v_ref):
    y = jnp.dot(x_ref[...], w_ref[...], preferred_element_type=jnp.float32)
    y = y + b_ref[...]
    q_ref[...] = y[:, :D]
    k_ref[...] = y[:, D:2 * D]
    v_ref[...] = y[:, 2 * D:]


_qkv_call = pl.pallas_call(
    _qkv_body,
    grid=(N // _RB,),
    in_specs=[
        pl.BlockSpec((_RB, D), lambda i: (i, 0)),
        pl.BlockSpec((D, 3 * D), lambda i: (0, 0)),
        pl.BlockSpec((1, 3 * D), lambda i: (0, 0)),
    ],
    out_specs=[
        pl.BlockSpec((_RB, D), lambda i: (i, 0)),
        pl.BlockSpec((_RB, D), lambda i: (i, 0)),
        pl.BlockSpec((_RB, D), lambda i: (i, 0)),
    ],
    out_shape=[
        jax.ShapeDtypeStruct((N, D), jnp.float32),
        jax.ShapeDtypeStruct((N, D), jnp.float32),
        jax.ShapeDtypeStruct((N, D), jnp.float32),
    ],
)


def _final_body(a0_ref, a1_ref, x_ref, w_ref, b_ref, al_ref, o_ref):
    agg = a0_ref[...] + a1_ref[...]
    trans = jnp.dot(agg, w_ref[...], preferred_element_type=jnp.float32)
    trans = trans + b_ref[...]
    alpha = al_ref[0, 0]
    o_ref[...] = trans * alpha + x_ref[...] * (1.0 - alpha)


_final_call = pl.pallas_call(
    _final_body,
    grid=(N // _RB,),
    in_specs=[
        pl.BlockSpec((_RB, D), lambda i: (i, 0)),
        pl.BlockSpec((_RB, D), lambda i: (i, 0)),
        pl.BlockSpec((_RB, D), lambda i: (i, 0)),
        pl.BlockSpec((D, D), lambda i: (0, 0)),
        pl.BlockSpec((1, D), lambda i: (0, 0)),
        pl.BlockSpec((1, 1), lambda i: (0, 0)),
    ],
    out_specs=pl.BlockSpec((_RB, D), lambda i: (i, 0)),
    out_shape=jax.ShapeDtypeStruct((N, D), jnp.float32),
)


def kernel(x, edge_index, Wk, bk, Wq, bq, Wv, bv, Wa, ba, rel_att, rel_msg,
           rel_pri, skip):
    src = edge_index[0]
    dst = edge_index[1]

    scale = rel_pri[0] / np.sqrt(DK)
    Wq_f = (Wq.reshape(D, H, DK) * scale[None, :, None]).reshape(D, D)
    bq_f = (bq.reshape(H, DK) * scale[:, None]).reshape(D)
    Wk_f = jnp.einsum('dhj,hjk->dhk', Wk.reshape(D, H, DK), rel_att[0]).reshape(D, D)
    bk_f = jnp.einsum('hj,hjk->hk', bk.reshape(H, DK), rel_att[0]).reshape(D)
    Wv_f = jnp.einsum('dhj,hjk->dhk', Wv.reshape(D, H, DK), rel_msg[0]).reshape(D, D)
    bv_f = jnp.einsum('hj,hjk->hk', bv.reshape(H, DK), rel_msg[0]).reshape(D)
    perm = np.array([(j % H) * DK + j // H for j in range(D)])
    Wcat = jnp.concatenate([Wq_f, Wk_f, Wv_f[:, perm]], axis=1)
    bcat = jnp.concatenate([bq_f, bk_f, bv_f[perm]]).reshape(1, 3 * D)

    q, k, v = _qkv_call(x, Wcat, bcat)

    znd = jnp.zeros((N, D), jnp.float32)

    p, den = _sc_scores(q, k, src, dst)
    agg0, agg1 = _sc_aggregate(v, src, dst, p, den, znd)

    alpha = jax.nn.sigmoid(skip[0]).reshape(1, 1)
    return _final_call(agg0, agg1, x, Wa[perm, :], ba.reshape(1, D), alpha)


# speedup vs baseline: 21.4540x; 1.1035x over previous
import functools

import jax
import jax.numpy as jnp
import numpy as np
from jax import lax
from jax.experimental import pallas as pl
from jax.experimental.pallas import tpu as pltpu
from jax.experimental.pallas import tpu_sc as plsc

N = 10000
E = 320000
D = 128
H = 8
DK = D // H

NC = 2
NS = 16
NW = NC * NS
CHUNK = 128
ECHUNKS = E // CHUNK
NCH_MAIN = ECHUNKS // NW
TAILC = ECHUNKS - NW * NCH_MAIN
SUB = CHUNK // 16
ROWS_PT = 624
TAIL = N - NS * ROWS_PT

NPN = 640
DROW = NPN // 16
DSZ = NPN * H

_mesh = plsc.VectorSubcoreMesh(
    core_axis_name="c", subcore_axis_name="s", num_cores=NC, num_subcores=NS
)
_sc_params = pltpu.CompilerParams(needs_layout_passes=False)


def _copy_rows(src, dst, sid):
    pltpu.sync_copy(src.at[pl.ds(sid * ROWS_PT, ROWS_PT)],
                    dst.at[pl.ds(sid * ROWS_PT, ROWS_PT)])

    @pl.when(sid == NS - 1)
    def _():
        pltpu.sync_copy(src.at[pl.ds(NS * ROWS_PT, TAIL)],
                        dst.at[pl.ds(NS * ROWS_PT, TAIL)])


def _iota16():
    return lax.iota(jnp.int32, 16)


def _full16(v):
    return jnp.full((16,), v, jnp.int32)


@functools.partial(
    pl.kernel,
    mesh=_mesh,
    out_type=[
        jax.ShapeDtypeStruct((ECHUNKS, H * CHUNK), jnp.float32),
        jax.ShapeDtypeStruct((NW * NPN, D), jnp.float32),
    ],
    scratch_types=[
        pltpu.VMEM((CHUNK,), jnp.int32),
        pltpu.VMEM((CHUNK,), jnp.int32),
        pltpu.VMEM((H * CHUNK,), jnp.float32),
        pltpu.VMEM((NPN * 16 * H,), jnp.float32),
        pltpu.VMEM((DROW, D), jnp.float32),
        pltpu.SemaphoreType.DMA,
        pltpu.SemaphoreType.DMA,
    ],
    compiler_params=_sc_params,
)
def _sc_scores(q_hbm, k_hbm, src_hbm, dst_hbm,
               p_hbm, den_hbm,
               sidx, didx, pblkT, dacc, dstage, sem1, sem2):
    cid = lax.axis_index("c")
    sid = lax.axis_index("s")
    wid = sid * NC + cid

    io = _iota16()
    zv = jnp.zeros((16,), jnp.float32)

    def zero_body(i, _):
        dacc[pl.ds(i * 16, 16)] = zv
        return 0

    lax.fori_loop(0, NPN * 16 * H // 16, zero_body, 0)

    def scoped(qblk, kblk):
        def chunk_work(g):
            base = g * CHUNK
            pltpu.sync_copy(src_hbm.at[pl.ds(base, CHUNK)], sidx)
            pltpu.sync_copy(dst_hbm.at[pl.ds(base, CHUNK)], didx)
            cp_q = pltpu.async_copy(q_hbm.at[didx], qblk, sem1)
            cp_k = pltpu.async_copy(k_hbm.at[sidx], kblk, sem2)
            cp_q.wait()
            cp_k.wait()

            def sub_body(b, _):
                rows = _full16(0) + io + b * 16
                dv = didx[pl.ds(b * 16, 16)] * H
                for h in range(H):
                    qs = [plsc.load_gather(qblk, [rows, _full16(h * DK + dk)])
                          for dk in range(DK)]
                    ks = [plsc.load_gather(kblk, [rows, _full16(h * DK + dk)])
                          for dk in range(DK)]
                    ps = [qs[dk] * ks[dk] for dk in range(DK)]
                    while len(ps) > 1:
                        ps = [ps[i] + ps[i + 1] for i in range(0, len(ps), 2)]
                    pexp = jnp.exp(ps[0])
                    pblkT[pl.ds(h * CHUNK + b * 16, 16)] = pexp
                    plsc.addupdate_scatter(dacc, [dv + h], pexp)
                return 0

            lax.fori_loop(0, SUB, sub_body, 0)
            pltpu.sync_copy(pblkT, p_hbm.at[g])

        def chunk_body(c, _):
            chunk_work(c * NW + wid)
            return 0

        lax.fori_loop(0, NCH_MAIN, chunk_body, 0)

        @pl.when(wid < TAILC)
        def _():
            chunk_work(NW * NCH_MAIN + wid)

    pl.run_scoped(scoped,
                  pltpu.VMEM((CHUNK, D), jnp.float32),
                  pltpu.VMEM((CHUNK, D), jnp.float32))

    def dump_piece(pc, _):
        def row_body(r, _):
            for j in range(8):
                dstage[r, pl.ds(j * 16, 16)] = (
                    dacc[pl.ds((pc * DROW + r) * D + j * 16, 16)])
            return 0

        lax.fori_loop(0, DROW, row_body, 0)
        pltpu.sync_copy(dstage,
                        den_hbm.at[pl.ds(wid * NPN + pc * DROW, DROW)])
        return 0

    lax.fori_loop(0, 16, dump_piece, 0)


@functools.partial(
    pl.kernel,
    mesh=_mesh,
    out_type=[
        jax.ShapeDtypeStruct((N, D), jnp.float32),
        jax.ShapeDtypeStruct((N, D), jnp.float32),
    ],
    scratch_types=[
        pltpu.VMEM((CHUNK,), jnp.int32),
        pltpu.VMEM((CHUNK,), jnp.int32),
        pltpu.VMEM((CHUNK,), jnp.int32),
        pltpu.VMEM((H * CHUNK,), jnp.float32),
        pltpu.VMEM((H * CHUNK,), jnp.float32),
        pltpu.VMEM((DROW, D), jnp.float32),
        pltpu.VMEM((DROW, D), jnp.float32),
        pltpu.VMEM_SHARED((NS * DROW, D), jnp.float32),
        pltpu.VMEM_SHARED((N, D), jnp.float32),
        pltpu.SemaphoreType.DMA,
        pltpu.SemaphoreType.DMA,
    ],
    compiler_params=_sc_params,
)
def _sc_aggregate(v_hbm, src_hbm, dst_hbm, p_hbm, den_hbm, znd_hbm,
                  agg0_hbm, agg1_hbm,
                  sidx, didx, didx2, pblkT, attnT, acc, stg,
                  rden_sh, agg_sh, sem1, sem2):
    cid = lax.axis_index("c")
    sid = lax.axis_index("s")
    wid = sid * NC + cid

    _copy_rows(znd_hbm, agg_sh, sid)

    io = _iota16()

    rowbase = sid * DROW

    def zero_body(r, _):
        for j in range(8):
            acc[r, pl.ds(j * 16, 16)] = jnp.zeros((16,), jnp.float32)
        return 0

    lax.fori_loop(0, DROW, zero_body, 0)

    def add_tile(t, _):
        pltpu.sync_copy(den_hbm.at[pl.ds(t * NPN + rowbase, DROW)], stg)

        def row_body(r, _):
            for j in range(8):
                sl = pl.ds(j * 16, 16)
                acc[r, sl] = acc[r, sl] + stg[r, sl]
            return 0

        lax.fori_loop(0, DROW, row_body, 0)
        return 0

    lax.fori_loop(0, NW, add_tile, 0)

    def rcp_body(r, _):
        for j in range(8):
            sl = pl.ds(j * 16, 16)
            acc[r, sl] = 1.0 / acc[r, sl]
        return 0

    lax.fori_loop(0, DROW, rcp_body, 0)
    pltpu.sync_copy(acc, rden_sh.at[pl.ds(rowbase, DROW)])
    plsc.subcore_barrier()

    def scoped(vblk, rdnblk):
        def chunk_work(g):
            base = g * CHUNK
            pltpu.sync_copy(src_hbm.at[pl.ds(base, CHUNK)], sidx)
            pltpu.sync_copy(dst_hbm.at[pl.ds(base, CHUNK)], didx)
            pltpu.sync_copy(p_hbm.at[g], pblkT)
            cp_v = pltpu.async_copy(v_hbm.at[sidx], vblk, sem1)

            def didx2_body(b, _):
                didx2[pl.ds(b * 16, 16)] = (
                    lax.shift_right_logical(didx[pl.ds(b * 16, 16)], 4))
                return 0

            lax.fori_loop(0, SUB, didx2_body, 0)
            cp_r = pltpu.async_copy(rden_sh.at[didx2], rdnblk, sem2)
            cp_v.wait()
            cp_r.wait()

            def sub_body(b, _):
                rows = _full16(0) + io + b * 16
                coff = (didx[pl.ds(b * 16, 16)] & 15) * H
                for h in range(H):
                    pcol = pblkT[pl.ds(h * CHUNK + b * 16, 16)]
                    rd = plsc.load_gather(rdnblk, [rows, coff + h])
                    attnT[pl.ds(h * CHUNK + b * 16, 16)] = pcol * rd
                hidx = (io & 7) * CHUNK + b * 16
                for l in range(16):
                    e = b * 16 + l
                    arep = plsc.load_gather(attnT, [hidx + l])
                    for j in range(H):
                        vblk[e, pl.ds(j * 16, 16)] = (
                            vblk[e, pl.ds(j * 16, 16)] * arep)
                return 0

            lax.fori_loop(0, SUB, sub_body, 0)
            pltpu.sync_copy(vblk, agg_sh.at[didx], add=True)

        def chunk_body(c, _):
            chunk_work(c * NW + wid)
            return 0

        lax.fori_loop(0, NCH_MAIN, chunk_body, 0)

        @pl.when(wid < TAILC)
        def _():
            chunk_work(NW * NCH_MAIN + wid)

    pl.run_scoped(scoped,
                  pltpu.VMEM((CHUNK, D), jnp.float32),
                  pltpu.VMEM((CHUNK, D), jnp.float32))
    plsc.subcore_barrier()

    @pl.when(cid == 0)
    def _():
        _copy_rows(agg_sh, agg0_hbm, sid)

    @pl.when(cid == 1)
    def _():
        _copy_rows(agg_sh, agg1_hbm, sid)


_RB = 1000


def _qkv_body(x_ref, w_ref, b_ref, q_ref, k_ref, v_ref):
    y = jnp.dot(x_ref[...], w_ref[...], preferred_element_type=jnp.float32)
    y = y + b_ref[...]
    q_ref[...] = y[:, :D]
    k_ref[...] = y[:, D:2 * D]
    v_ref[...] = y[:, 2 * D:]


_qkv_call = pl.pallas_call(
    _qkv_body,
    grid=(N // _RB,),
    in_specs=[
        pl.BlockSpec((_RB, D), lambda i: (i, 0)),
        pl.BlockSpec((D, 3 * D), lambda i: (0, 0)),
        pl.BlockSpec((1, 3 * D), lambda i: (0, 0)),
    ],
    out_specs=[
        pl.BlockSpec((_RB, D), lambda i: (i, 0)),
        pl.BlockSpec((_RB, D), lambda i: (i, 0)),
        pl.BlockSpec((_RB, D), lambda i: (i, 0)),
    ],
    out_shape=[
        jax.ShapeDtypeStruct((N, D), jnp.float32),
        jax.ShapeDtypeStruct((N, D), jnp.float32),
        jax.ShapeDtypeStruct((N, D), jnp.float32),
    ],
)


def _final_body(a0_ref, a1_ref, x_ref, w_ref, b_ref, al_ref, o_ref):
    agg = a0_ref[...] + a1_ref[...]
    trans = jnp.dot(agg, w_ref[...], preferred_element_type=jnp.float32)
    trans = trans + b_ref[...]
    alpha = al_ref[0, 0]
    o_ref[...] = trans * alpha + x_ref[...] * (1.0 - alpha)


_final_call = pl.pallas_call(
    _final_body,
    grid=(N // _RB,),
    in_specs=[
        pl.BlockSpec((_RB, D), lambda i: (i, 0)),
        pl.BlockSpec((_RB, D), lambda i: (i, 0)),
        pl.BlockSpec((_RB, D), lambda i: (i, 0)),
        pl.BlockSpec((D, D), lambda i: (0, 0)),
        pl.BlockSpec((1, D), lambda i: (0, 0)),
        pl.BlockSpec((1, 1), lambda i: (0, 0)),
    ],
    out_specs=pl.BlockSpec((_RB, D), lambda i: (i, 0)),
    out_shape=jax.ShapeDtypeStruct((N, D), jnp.float32),
)


def kernel(x, edge_index, Wk, bk, Wq, bq, Wv, bv, Wa, ba, rel_att, rel_msg,
           rel_pri, skip):
    src = edge_index[0]
    dst = edge_index[1]

    scale = rel_pri[0] / np.sqrt(DK)
    Wq_f = (Wq.reshape(D, H, DK) * scale[None, :, None]).reshape(D, D)
    bq_f = (bq.reshape(H, DK) * scale[:, None]).reshape(D)
    Wk_f = jnp.einsum('dhj,hjk->dhk', Wk.reshape(D, H, DK), rel_att[0]).reshape(D, D)
    bk_f = jnp.einsum('hj,hjk->hk', bk.reshape(H, DK), rel_att[0]).reshape(D)
    Wv_f = jnp.einsum('dhj,hjk->dhk', Wv.reshape(D, H, DK), rel_msg[0]).reshape(D, D)
    bv_f = jnp.einsum('hj,hjk->hk', bv.reshape(H, DK), rel_msg[0]).reshape(D)
    perm = np.array([(j % H) * DK + j // H for j in range(D)])
    Wcat = jnp.concatenate([Wq_f, Wk_f, Wv_f[:, perm]], axis=1)
    bcat = jnp.concatenate([bq_f, bk_f, bv_f[perm]]).reshape(1, 3 * D)

    q, k, v = _qkv_call(x, Wcat, bcat)

    znd = jnp.zeros((N, D), jnp.float32)

    p, den = _sc_scores(q, k, src, dst)
    agg0, agg1 = _sc_aggregate(v, src, dst, p, den, znd)

    alpha = jax.nn.sigmoid(skip[0]).reshape(1, 1)
    return _final_call(agg0, agg1, x, Wa[perm, :], ba.reshape(1, D), alpha)
